# baseline (speedup 1.0000x reference)
import functools

import jax
import jax.numpy as jnp
from jax import lax
from jax.experimental import pallas as pl
from jax.experimental.pallas import tpu as pltpu
from jax.experimental.pallas import tpu_sc as plsc

N = 10000
E = 320000
D = 128
H = 128
C = 10
G = 64
NUM_LAYERS = 3
BN_EPS = 1e-5

NC = 2
NS = 16
NW = NC * NS
CHUNK = 50
CHUNKS = 200
AGG_N = N
GRP = 40
NGRP = CHUNKS // GRP
NBUF = 4
RPS = 624
REM = N - NS * RPS
BLK = 1000
C_PAD = 128


def _agg_body(h_hbm, eidx_hbm, out_hbm, srcb, dstb, rows, agg_sh,
              g0, g1, g2, g3, s0, s1, s2, s3, isem):
    gsems = (g0, g1, g2, g3)
    ssems = (s0, s1, s2, s3)
    c = lax.axis_index("c")
    s = lax.axis_index("s")
    wid = s * NC + c

    pltpu.sync_copy(h_hbm.at[pl.ds(s * RPS, RPS)], agg_sh.at[pl.ds(s * RPS, RPS)])

    @pl.when(s == NS - 1)
    def _():
        pltpu.sync_copy(h_hbm.at[pl.ds(NS * RPS, REM)],
                        agg_sh.at[pl.ds(NS * RPS, REM)])

    my_src = eidx_hbm.at[0, wid]
    my_dst = eidx_hbm.at[1, wid]
    pltpu.sync_copy(my_src.at[pl.ds(0, GRP)], srcb.at[0])
    pltpu.sync_copy(my_dst.at[pl.ds(0, GRP)], dstb.at[0])
    plsc.subcore_barrier()

    LOOK = NBUF - 1

    def wait_gather(bi, ki, r):
        pltpu.make_async_copy(h_hbm.at[srcb.at[bi, ki]], rows.at[r],
                              gsems[r]).wait()

    def wait_scatter(bi, ki, r):
        pltpu.make_async_copy(rows.at[r], agg_sh.at[dstb.at[bi, ki]],
                              ssems[r]).wait()

    for k in range(LOOK):
        pltpu.async_copy(h_hbm.at[srcb.at[0, k]], rows.at[k], gsems[k])

    def group(g, carry):
        b = lax.rem(g, 2)
        nb = 1 - b

        @pl.when(g < NGRP - 1)
        def _():
            pltpu.async_copy(my_src.at[pl.ds((g + 1) * GRP, GRP)],
                             srcb.at[nb], isem)
            pltpu.async_copy(my_dst.at[pl.ds((g + 1) * GRP, GRP)],
                             dstb.at[nb], isem)

        for k in range(GRP):
            r = k % NBUF
            rl = (k + LOOK) % NBUF

            wait_gather(b, k, r)
            pltpu.async_copy(rows.at[r], agg_sh.at[dstb.at[b, k]], ssems[r],
                             add=True)

            if k < GRP - LOOK:
                if k == 0:
                    @pl.when(g > 0)
                    def _():
                        wait_scatter(b, GRP - 1, rl)
                else:
                    wait_scatter(b, k - 1, rl)
                pltpu.async_copy(h_hbm.at[srcb.at[b, k + LOOK]], rows.at[rl],
                                 gsems[rl])
            else:
                if k == GRP - LOOK:
                    @pl.when(g < NGRP - 1)
                    def _():
                        pltpu.make_async_copy(
                            my_src.at[pl.ds((g + 1) * GRP, GRP)],
                            srcb.at[nb], isem).wait()
                        pltpu.make_async_copy(
                            my_dst.at[pl.ds((g + 1) * GRP, GRP)],
                            dstb.at[nb], isem).wait()

                @pl.when(g < NGRP - 1)
                def _():
                    wait_scatter(b, k - 1, rl)
                    pltpu.async_copy(h_hbm.at[srcb.at[nb, k - (GRP - LOOK)]],
                                     rows.at[rl], gsems[rl])
        return carry

    lax.fori_loop(0, NGRP, group, 0)

    bl = (NGRP - 1) % 2
    for k in range(GRP - NBUF, GRP):
        wait_scatter(bl, k, k % NBUF)
    plsc.subcore_barrier()

    pltpu.sync_copy(agg_sh.at[pl.ds(s * RPS, RPS)],
                    out_hbm.at[c].at[pl.ds(s * RPS, RPS)])

    @pl.when(s == NS - 1)
    def _():
        pltpu.sync_copy(agg_sh.at[pl.ds(NS * RPS, REM)],
                        out_hbm.at[c].at[pl.ds(NS * RPS, REM)])


@functools.cache
def _make_agg_call():
    return pl.kernel(
        _agg_body,
        out_type=jax.ShapeDtypeStruct((NC, N, D), jnp.float32),
        mesh=plsc.VectorSubcoreMesh(core_axis_name="c", subcore_axis_name="s"),
        scratch_types=[
            pltpu.VMEM((2, GRP, CHUNK), jnp.int32),
            pltpu.VMEM((2, GRP, CHUNK), jnp.int32),
            pltpu.VMEM((NBUF, CHUNK, D), jnp.float32),
            pltpu.VMEM_SHARED((AGG_N, D), jnp.float32),
        ] + [pltpu.SemaphoreType.DMA] * (2 * NBUF + 1),
    )


def _mlp_body(parts_ref, h_ref, eps_ref, w1_ref, b1_ref, w2_ref, b2_ref,
              out_ref):
    z = parts_ref[0] + parts_ref[1] + eps_ref[0, 0] * h_ref[...]
    a = jnp.dot(z, w1_ref[...], preferred_element_type=jnp.float32)
    a = jnp.maximum(a + b1_ref[...], 0.0)
    o = jnp.dot(a, w2_ref[...], preferred_element_type=jnp.float32)
    out_ref[...] = jnp.maximum(o + b2_ref[...], 0.0)


def _mlp_call(parts, h, epsm1, w1, b1, w2, b2):
    grid = (N // BLK,)
    return pl.pallas_call(
        _mlp_body,
        grid=grid,
        in_specs=[
            pl.BlockSpec((NC, BLK, D), lambda i: (0, i, 0)),
            pl.BlockSpec((BLK, D), lambda i: (i, 0)),
            pl.BlockSpec((1, 1), lambda i: (0, 0)),
            pl.BlockSpec((D, H), lambda i: (0, 0)),
            pl.BlockSpec((1, H), lambda i: (0, 0)),
            pl.BlockSpec((H, H), lambda i: (0, 0)),
            pl.BlockSpec((1, H), lambda i: (0, 0)),
        ],
        out_specs=pl.BlockSpec((BLK, H), lambda i: (i, 0)),
        out_shape=jax.ShapeDtypeStruct((N, H), jnp.float32),
    )(parts, h, epsm1, w1, b1, w2, b2)


def _final_body(parts_ref, h_ref, eps_ref, w1_ref, b1_ref, w2_ref, b2_ref,
                batch_ref, l1w_ref, l1b_ref, l2w_ref, l2b_ref, out_ref,
                pooled_ref):
    i = pl.program_id(0)
    z = parts_ref[0] + parts_ref[1] + eps_ref[0, 0] * h_ref[...]
    a = jnp.dot(z, w1_ref[...], preferred_element_type=jnp.float32)
    a = jnp.maximum(a + b1_ref[...], 0.0)
    h3 = jnp.dot(a, w2_ref[...], preferred_element_type=jnp.float32)
    h3 = jnp.maximum(h3 + b2_ref[...], 0.0)

    seg = batch_ref[...]
    onehot = (seg == lax.broadcasted_iota(jnp.int32, (BLK, G), 1))
    onehot = onehot.astype(jnp.float32)
    contrib = lax.dot_general(onehot, h3, (((0,), (0,)), ((), ())),
                              preferred_element_type=jnp.float32)

    @pl.when(i == 0)
    def _():
        pooled_ref[...] = contrib

    @pl.when(i > 0)
    def _():
        pooled_ref[...] += contrib

    @pl.when(i == pl.num_programs(0) - 1)
    def _():
        t = jnp.dot(pooled_ref[...], l1w_ref[...],
                    preferred_element_type=jnp.float32)
        t = jnp.maximum(t + l1b_ref[...], 0.0)
        o = jnp.dot(t, l2w_ref[...], preferred_element_type=jnp.float32)
        o = o + l2b_ref[...]
        m = jnp.max(o, axis=1, keepdims=True)
        lse = jnp.log(jnp.sum(jnp.exp(o - m), axis=1, keepdims=True)) + m
        out_ref[...] = o - lse


def _final_call(parts, h, epsm1, w1, b1, w2, b2, batch2d, l1w, l1b, l2w, l2b):
    grid = (N // BLK,)
    return pl.pallas_call(
        _final_body,
        grid=grid,
        in_specs=[
            pl.BlockSpec((NC, BLK, D), lambda i: (0, i, 0)),
            pl.BlockSpec((BLK, D), lambda i: (i, 0)),
            pl.BlockSpec((1, 1), lambda i: (0, 0)),
            pl.BlockSpec((D, H), lambda i: (0, 0)),
            pl.BlockSpec((1, H), lambda i: (0, 0)),
            pl.BlockSpec((H, H), lambda i: (0, 0)),
            pl.BlockSpec((1, H), lambda i: (0, 0)),
            pl.BlockSpec((BLK, 1), lambda i: (i, 0)),
            pl.BlockSpec((H, H), lambda i: (0, 0)),
            pl.BlockSpec((1, H), lambda i: (0, 0)),
            pl.BlockSpec((H, C_PAD), lambda i: (0, 0)),
            pl.BlockSpec((1, C_PAD), lambda i: (0, 0)),
        ],
        out_specs=pl.BlockSpec((G, C_PAD), lambda i: (0, 0)),
        out_shape=jax.ShapeDtypeStruct((G, C_PAD), jnp.float32),
        scratch_shapes=[pltpu.VMEM((G, H), jnp.float32)],
    )(parts, h, epsm1, w1, b1, w2, b2, batch2d, l1w, l1b, l2w, l2b)


def kernel(x, edge_index, batch, params):
    bn_scale = 1.0 / jnp.sqrt(1.0 + BN_EPS)

    eidx = edge_index.reshape(2, NW, CHUNKS, CHUNK)
    batch2d = batch.reshape(N, 1)

    folded = []
    for i in range(NUM_LAYERS):
        s1 = bn_scale * params[f"g1_{i}"]
        w1 = params[f"W1_{i}"] * s1[None, :]
        b1 = (params[f"b1_{i}"] * s1 + params[f"bt1_{i}"]).reshape(1, H)
        s2 = bn_scale * params[f"g_{i}"]
        w2 = params[f"W2_{i}"] * s2[None, :]
        b2 = (params[f"b2_{i}"] * s2 + params[f"bt_{i}"]).reshape(1, H)
        epsm1 = (params[f"eps_{i}"] - 1.0).reshape(1, 1).astype(jnp.float32)
        folded.append((epsm1, w1, b1, w2, b2))

    l2w = jnp.zeros((H, C_PAD), jnp.float32).at[:, :C].set(params["lin2_W"])
    l2b = jnp.full((1, C_PAD), -1e30, jnp.float32).at[0, :C].set(params["lin2_b"])
    l1b = params["lin1_b"].reshape(1, H)

    h = x
    for i in range(NUM_LAYERS):
        parts = _make_agg_call()(h, eidx)
        epsm1, w1, b1, w2, b2 = folded[i]
        if i < NUM_LAYERS - 1:
            h = _mlp_call(parts, h, epsm1, w1, b1, w2, b2)
        else:
            out = _final_call(parts, h, epsm1, w1, b1, w2, b2, batch2d,
                              params["lin1_W"], l1b, l2w, l2b)
    return out[:, :C]


# speedup vs baseline: 1.0238x; 1.0238x over previous
import functools

import jax
import jax.numpy as jnp
from jax import lax
from jax.experimental import pallas as pl
from jax.experimental.pallas import tpu as pltpu
from jax.experimental.pallas import tpu_sc as plsc

N = 10000
E = 320000
D = 128
H = 128
C = 10
G = 64
NUM_LAYERS = 3
BN_EPS = 1e-5

NC = 2
NS = 16
NW = NC * NS
CHUNK = 50
CHUNKS = 200
AGG_N = N
GRP = 40
NGRP = CHUNKS // GRP
NBUF = 4
RPS = 624
REM = N - NS * RPS
BLK = 2000
C_PAD = 128


def _agg_body(h_hbm, eidx_hbm, out_hbm, srcb, dstb, rows, agg_sh,
              g0, g1, g2, g3, s0, s1, s2, s3, isem):
    gsems = (g0, g1, g2, g3)
    ssems = (s0, s1, s2, s3)
    c = lax.axis_index("c")
    s = lax.axis_index("s")
    wid = s * NC + c

    pltpu.sync_copy(h_hbm.at[pl.ds(s * RPS, RPS)], agg_sh.at[pl.ds(s * RPS, RPS)])

    @pl.when(s == NS - 1)
    def _():
        pltpu.sync_copy(h_hbm.at[pl.ds(NS * RPS, REM)],
                        agg_sh.at[pl.ds(NS * RPS, REM)])

    my_src = eidx_hbm.at[0, wid]
    my_dst = eidx_hbm.at[1, wid]
    pltpu.sync_copy(my_src.at[pl.ds(0, GRP)], srcb.at[0])
    pltpu.sync_copy(my_dst.at[pl.ds(0, GRP)], dstb.at[0])
    plsc.subcore_barrier()

    LOOK = NBUF - 1

    def wait_gather(bi, ki, r):
        pltpu.make_async_copy(h_hbm.at[srcb.at[bi, ki]], rows.at[r],
                              gsems[r]).wait()

    def wait_scatter(bi, ki, r):
        pltpu.make_async_copy(rows.at[r], agg_sh.at[dstb.at[bi, ki]],
                              ssems[r]).wait()

    for k in range(LOOK):
        pltpu.async_copy(h_hbm.at[srcb.at[0, k]], rows.at[k], gsems[k])

    def group(g, carry):
        b = lax.rem(g, 2)
        nb = 1 - b

        @pl.when(g < NGRP - 1)
        def _():
            pltpu.async_copy(my_src.at[pl.ds((g + 1) * GRP, GRP)],
                             srcb.at[nb], isem)
            pltpu.async_copy(my_dst.at[pl.ds((g + 1) * GRP, GRP)],
                             dstb.at[nb], isem)

        for k in range(GRP):
            r = k % NBUF
            rl = (k + LOOK) % NBUF

            wait_gather(b, k, r)
            pltpu.async_copy(rows.at[r], agg_sh.at[dstb.at[b, k]], ssems[r],
                             add=True)

            if k < GRP - LOOK:
                if k == 0:
                    @pl.when(g > 0)
                    def _():
                        wait_scatter(b, GRP - 1, rl)
                else:
                    wait_scatter(b, k - 1, rl)
                pltpu.async_copy(h_hbm.at[srcb.at[b, k + LOOK]], rows.at[rl],
                                 gsems[rl])
            else:
                if k == GRP - LOOK:
                    @pl.when(g < NGRP - 1)
                    def _():
                        pltpu.make_async_copy(
                            my_src.at[pl.ds((g + 1) * GRP, GRP)],
                            srcb.at[nb], isem).wait()
                        pltpu.make_async_copy(
                            my_dst.at[pl.ds((g + 1) * GRP, GRP)],
                            dstb.at[nb], isem).wait()

                @pl.when(g < NGRP - 1)
                def _():
                    wait_scatter(b, k - 1, rl)
                    pltpu.async_copy(h_hbm.at[srcb.at[nb, k - (GRP - LOOK)]],
                                     rows.at[rl], gsems[rl])
        return carry

    lax.fori_loop(0, NGRP, group, 0)

    bl = (NGRP - 1) % 2
    for k in range(GRP - NBUF, GRP):
        wait_scatter(bl, k, k % NBUF)
    plsc.subcore_barrier()

    pltpu.sync_copy(agg_sh.at[pl.ds(s * RPS, RPS)],
                    out_hbm.at[c].at[pl.ds(s * RPS, RPS)])

    @pl.when(s == NS - 1)
    def _():
        pltpu.sync_copy(agg_sh.at[pl.ds(NS * RPS, REM)],
                        out_hbm.at[c].at[pl.ds(NS * RPS, REM)])


@functools.cache
def _make_agg_call():
    return pl.kernel(
        _agg_body,
        out_type=jax.ShapeDtypeStruct((NC, N, D), jnp.float32),
        mesh=plsc.VectorSubcoreMesh(core_axis_name="c", subcore_axis_name="s"),
        scratch_types=[
            pltpu.VMEM((2, GRP, CHUNK), jnp.int32),
            pltpu.VMEM((2, GRP, CHUNK), jnp.int32),
            pltpu.VMEM((NBUF, CHUNK, D), jnp.float32),
            pltpu.VMEM_SHARED((AGG_N, D), jnp.float32),
        ] + [pltpu.SemaphoreType.DMA] * (2 * NBUF + 1),
    )


def _mlp_body(parts_ref, h_ref, eps_ref, w1_ref, b1_ref, w2_ref, b2_ref,
              out_ref):
    z = parts_ref[0] + parts_ref[1] + eps_ref[0, 0] * h_ref[...]
    a = jnp.dot(z, w1_ref[...], preferred_element_type=jnp.float32)
    a = jnp.maximum(a + b1_ref[...], 0.0)
    o = jnp.dot(a, w2_ref[...], preferred_element_type=jnp.float32)
    out_ref[...] = jnp.maximum(o + b2_ref[...], 0.0)


def _mlp_call(parts, h, epsm1, w1, b1, w2, b2):
    grid = (N // BLK,)
    return pl.pallas_call(
        _mlp_body,
        grid=grid,
        in_specs=[
            pl.BlockSpec((NC, BLK, D), lambda i: (0, i, 0)),
            pl.BlockSpec((BLK, D), lambda i: (i, 0)),
            pl.BlockSpec((1, 1), lambda i: (0, 0)),
            pl.BlockSpec((D, H), lambda i: (0, 0)),
            pl.BlockSpec((1, H), lambda i: (0, 0)),
            pl.BlockSpec((H, H), lambda i: (0, 0)),
            pl.BlockSpec((1, H), lambda i: (0, 0)),
        ],
        out_specs=pl.BlockSpec((BLK, H), lambda i: (i, 0)),
        out_shape=jax.ShapeDtypeStruct((N, H), jnp.float32),
    )(parts, h, epsm1, w1, b1, w2, b2)


def _final_body(parts_ref, h_ref, eps_ref, w1_ref, b1_ref, w2_ref, b2_ref,
                batch_ref, l1w_ref, l1b_ref, l2w_ref, l2b_ref, out_ref,
                pooled_ref):
    i = pl.program_id(0)
    z = parts_ref[0] + parts_ref[1] + eps_ref[0, 0] * h_ref[...]
    a = jnp.dot(z, w1_ref[...], preferred_element_type=jnp.float32)
    a = jnp.maximum(a + b1_ref[...], 0.0)
    h3 = jnp.dot(a, w2_ref[...], preferred_element_type=jnp.float32)
    h3 = jnp.maximum(h3 + b2_ref[...], 0.0)

    seg = batch_ref[...]
    onehot = (seg == lax.broadcasted_iota(jnp.int32, (BLK, G), 1))
    onehot = onehot.astype(jnp.float32)
    contrib = lax.dot_general(onehot, h3, (((0,), (0,)), ((), ())),
                              preferred_element_type=jnp.float32)

    @pl.when(i == 0)
    def _():
        pooled_ref[...] = contrib

    @pl.when(i > 0)
    def _():
        pooled_ref[...] += contrib

    @pl.when(i == pl.num_programs(0) - 1)
    def _():
        t = jnp.dot(pooled_ref[...], l1w_ref[...],
                    preferred_element_type=jnp.float32)
        t = jnp.maximum(t + l1b_ref[...], 0.0)
        o = jnp.dot(t, l2w_ref[...], preferred_element_type=jnp.float32)
        o = o + l2b_ref[...]
        m = jnp.max(o, axis=1, keepdims=True)
        lse = jnp.log(jnp.sum(jnp.exp(o - m), axis=1, keepdims=True)) + m
        out_ref[...] = o - lse


def _final_call(parts, h, epsm1, w1, b1, w2, b2, batch2d, l1w, l1b, l2w, l2b):
    grid = (N // BLK,)
    return pl.pallas_call(
        _final_body,
        grid=grid,
        in_specs=[
            pl.BlockSpec((NC, BLK, D), lambda i: (0, i, 0)),
            pl.BlockSpec((BLK, D), lambda i: (i, 0)),
            pl.BlockSpec((1, 1), lambda i: (0, 0)),
            pl.BlockSpec((D, H), lambda i: (0, 0)),
            pl.BlockSpec((1, H), lambda i: (0, 0)),
            pl.BlockSpec((H, H), lambda i: (0, 0)),
            pl.BlockSpec((1, H), lambda i: (0, 0)),
            pl.BlockSpec((BLK, 1), lambda i: (i, 0)),
            pl.BlockSpec((H, H), lambda i: (0, 0)),
            pl.BlockSpec((1, H), lambda i: (0, 0)),
            pl.BlockSpec((H, C_PAD), lambda i: (0, 0)),
            pl.BlockSpec((1, C_PAD), lambda i: (0, 0)),
        ],
        out_specs=pl.BlockSpec((G, C_PAD), lambda i: (0, 0)),
        out_shape=jax.ShapeDtypeStruct((G, C_PAD), jnp.float32),
        scratch_shapes=[pltpu.VMEM((G, H), jnp.float32)],
    )(parts, h, epsm1, w1, b1, w2, b2, batch2d, l1w, l1b, l2w, l2b)


def kernel(x, edge_index, batch, params):
    bn_scale = 1.0 / jnp.sqrt(1.0 + BN_EPS)

    eidx = edge_index.reshape(2, NW, CHUNKS, CHUNK)
    batch2d = batch.reshape(N, 1)

    folded = []
    for i in range(NUM_LAYERS):
        s1 = bn_scale * params[f"g1_{i}"]
        w1 = params[f"W1_{i}"] * s1[None, :]
        b1 = (params[f"b1_{i}"] * s1 + params[f"bt1_{i}"]).reshape(1, H)
        s2 = bn_scale * params[f"g_{i}"]
        w2 = params[f"W2_{i}"] * s2[None, :]
        b2 = (params[f"b2_{i}"] * s2 + params[f"bt_{i}"]).reshape(1, H)
        epsm1 = (params[f"eps_{i}"] - 1.0).reshape(1, 1).astype(jnp.float32)
        folded.append((epsm1, w1, b1, w2, b2))

    l2w = jnp.zeros((H, C_PAD), jnp.float32).at[:, :C].set(params["lin2_W"])
    l2b = jnp.full((1, C_PAD), -1e30, jnp.float32).at[0, :C].set(params["lin2_b"])
    l1b = params["lin1_b"].reshape(1, H)

    h = x
    for i in range(NUM_LAYERS):
        parts = _make_agg_call()(h, eidx)
        epsm1, w1, b1, w2, b2 = folded[i]
        if i < NUM_LAYERS - 1:
            h = _mlp_call(parts, h, epsm1, w1, b1, w2, b2)
        else:
            out = _final_call(parts, h, epsm1, w1, b1, w2, b2, batch2d,
                              params["lin1_W"], l1b, l2w, l2b)
    return out[:, :C]
